# baseline (speedup 1.0000x reference)
import functools

import jax
import jax.numpy as jnp
from jax import lax
from jax.experimental import pallas as pl
from jax.experimental.pallas import tpu as pltpu, tpu_sc as plsc

_N_TOKENS = 65536
_NUM_CODES = 1024
_CODE_DIM = 32
_BETA = 0.25
_BLK = 256
_CHUNK = 128
_NCHUNKS = _NUM_CODES // _CHUNK
_GCHUNK = 512


def _vq_body(z_ref, c_ref, zsq_ref, csq_ref, idx_ref, idxe_ref, part_ref):
    z = z_ref[:]
    c = c_ref[:]
    dots2 = jax.lax.dot_general(
        z * (-2.0), c, (((1,), (1,)), ((), ())),
        preferred_element_type=jnp.float32)
    zsq = zsq_ref[:][:, None]
    csq = csq_ref[:]

    best_val = (zsq + csq[None, 0:_CHUNK]) + dots2[:, 0:_CHUNK]
    best_k = jnp.zeros((_BLK, _CHUNK), jnp.float32)
    for k in range(1, _NCHUNKS):
        lo = k * _CHUNK
        dk = (zsq + csq[None, lo:lo + _CHUNK]) + dots2[:, lo:lo + _CHUNK]
        m = dk < best_val
        best_val = jnp.where(m, dk, best_val)
        best_k = jnp.where(m, float(k), best_k)

    gmin = jnp.min(best_val, axis=1, keepdims=True)
    lane = jax.lax.broadcasted_iota(
        jnp.int32, (_BLK, _CHUNK), 1).astype(jnp.float32)
    orig = best_k * float(_CHUNK) + lane
    cand = jnp.where(best_val == gmin, orig, float(_NUM_CODES))
    idxf = jnp.min(cand, axis=1)
    idx_ref[:] = idxf.astype(jnp.int32)
    idxe_ref[:] = (
        idxf[:, None] * float(_CODE_DIM)
        + jax.lax.broadcasted_iota(
            jnp.int32, (_BLK, _CODE_DIM), 1).astype(jnp.float32)
    ).astype(jnp.int32)
    part_ref[...] = jnp.sum(gmin).reshape(1, 1, 1)


def _sc_gather(codebook_flat, idxe):
    n_el = _N_TOKENS * _CODE_DIM
    info = plsc.get_sparse_core_info()
    nw = info.num_cores * info.num_subcores
    e_per_w = n_el // nw
    chunk = _GCHUNK * _CODE_DIM
    mesh = plsc.VectorSubcoreMesh(core_axis_name="c", subcore_axis_name="s")

    @functools.partial(
        pl.kernel, mesh=mesh,
        out_type=jax.ShapeDtypeStruct((n_el,), jnp.float32),
        scratch_types=[
            pltpu.VMEM((chunk,), jnp.int32),
            pltpu.VMEM((chunk,), jnp.float32),
            pltpu.VMEM_SHARED((_NUM_CODES * _CODE_DIM,), jnp.float32),
            pltpu.SemaphoreType.DMA,
        ],
    )
    def k(table_hbm, idx_hbm, out_hbm, idx_v, rows_v, cb_sp, sem):
        sid = lax.axis_index("s")
        wid = sid * info.num_cores + lax.axis_index("c")
        base = wid * e_per_w

        @pl.when(sid == 0)
        def _stage():
            pltpu.sync_copy(table_hbm, cb_sp)

        plsc.subcore_barrier()
        for j in range(e_per_w // chunk):
            lo = base + j * chunk
            pltpu.sync_copy(idx_hbm.at[pl.ds(lo, chunk)], idx_v)
            pltpu.async_copy(cb_sp.at[idx_v], rows_v, sem).wait()
            pltpu.sync_copy(rows_v, out_hbm.at[pl.ds(lo, chunk)])

    return k(codebook_flat, idxe)


@jax.jit
def kernel(z, codebook):
    zsq = jnp.sum(z ** 2, axis=1)
    csq = jnp.sum(codebook ** 2, axis=1)
    grid = _N_TOKENS // _BLK
    idx, idxe, part = pl.pallas_call(
        _vq_body,
        grid=(grid,),
        in_specs=[
            pl.BlockSpec((_BLK, _CODE_DIM), lambda i: (i, 0)),
            pl.BlockSpec((_NUM_CODES, _CODE_DIM), lambda i: (0, 0)),
            pl.BlockSpec((_BLK,), lambda i: (i,)),
            pl.BlockSpec((_NUM_CODES,), lambda i: (0,)),
        ],
        out_specs=[
            pl.BlockSpec((_BLK,), lambda i: (i,)),
            pl.BlockSpec((_BLK, _CODE_DIM), lambda i: (i, 0)),
            pl.BlockSpec((1, 1, 1), lambda i: (i, 0, 0)),
        ],
        out_shape=[
            jax.ShapeDtypeStruct((_N_TOKENS,), jnp.int32),
            jax.ShapeDtypeStruct((_N_TOKENS, _CODE_DIM), jnp.int32),
            jax.ShapeDtypeStruct((grid, 1, 1), jnp.float32),
        ],
        compiler_params=pltpu.CompilerParams(
            dimension_semantics=("parallel",)),
    )(z, codebook, zsq, csq)
    zq = _sc_gather(codebook.reshape(_NUM_CODES * _CODE_DIM),
                    idxe.reshape(_N_TOKENS * _CODE_DIM)
                    ).reshape(_N_TOKENS, _CODE_DIM)
    m = jnp.sum(part) / (_N_TOKENS * _CODE_DIM)
    loss = _BETA * m + m
    return (zq, idx, loss)


# speedup vs baseline: 1.2295x; 1.2295x over previous
import functools

import jax
import jax.numpy as jnp
from jax import lax
from jax.experimental import pallas as pl
from jax.experimental.pallas import tpu as pltpu, tpu_sc as plsc

_N_TOKENS = 65536
_NUM_CODES = 1024
_CODE_DIM = 32
_BETA = 0.25
_BLK = 512
_CHUNK = 128
_NCHUNKS = _NUM_CODES // _CHUNK
_GCHUNK = 1024


def _vq_body(z_ref, c_ref, zsq_ref, csq_ref, idx_ref, idxe_ref, part_ref):
    z = z_ref[:]
    c = c_ref[:]
    dots2 = jax.lax.dot_general(
        z * (-2.0), c, (((1,), (1,)), ((), ())),
        preferred_element_type=jnp.float32)
    zsq = zsq_ref[:][:, None]
    csq = csq_ref[:]

    best_val = (zsq + csq[None, 0:_CHUNK]) + dots2[:, 0:_CHUNK]
    best_k = jnp.zeros((_BLK, _CHUNK), jnp.float32)
    for k in range(1, _NCHUNKS):
        lo = k * _CHUNK
        dk = (zsq + csq[None, lo:lo + _CHUNK]) + dots2[:, lo:lo + _CHUNK]
        m = dk < best_val
        best_val = jnp.where(m, dk, best_val)
        best_k = jnp.where(m, float(k), best_k)

    gmin = jnp.min(best_val, axis=1, keepdims=True)
    lane = jax.lax.broadcasted_iota(
        jnp.int32, (_BLK, _CHUNK), 1).astype(jnp.float32)
    orig = best_k * float(_CHUNK) + lane
    cand = jnp.where(best_val == gmin, orig, float(_NUM_CODES))
    idxf = jnp.min(cand, axis=1)
    idx_ref[:] = idxf.astype(jnp.int32)
    idxe_ref[:] = (
        idxf[:, None] * float(_CODE_DIM)
        + jax.lax.broadcasted_iota(
            jnp.int32, (_BLK, _CODE_DIM), 1).astype(jnp.float32)
    ).astype(jnp.int32)
    part_ref[...] = jnp.sum(gmin).reshape(1, 1, 1)


def _sc_gather(codebook_flat, idxe):
    n_el = _N_TOKENS * _CODE_DIM
    info = plsc.get_sparse_core_info()
    nw = info.num_cores * info.num_subcores
    e_per_w = n_el // nw
    chunk = _GCHUNK * _CODE_DIM
    mesh = plsc.VectorSubcoreMesh(core_axis_name="c", subcore_axis_name="s")

    @functools.partial(
        pl.kernel, mesh=mesh,
        out_type=jax.ShapeDtypeStruct((n_el,), jnp.float32),
        scratch_types=[
            pltpu.VMEM((chunk,), jnp.int32),
            pltpu.VMEM((chunk,), jnp.float32),
            pltpu.VMEM_SHARED((_NUM_CODES * _CODE_DIM,), jnp.float32),
            pltpu.SemaphoreType.DMA,
        ],
    )
    def k(table_hbm, idx_hbm, out_hbm, idx_v, rows_v, cb_sp, sem):
        sid = lax.axis_index("s")
        wid = sid * info.num_cores + lax.axis_index("c")
        base = wid * e_per_w

        @pl.when(sid == 0)
        def _stage():
            pltpu.sync_copy(table_hbm, cb_sp)

        plsc.subcore_barrier()
        for j in range(e_per_w // chunk):
            lo = base + j * chunk
            pltpu.sync_copy(idx_hbm.at[pl.ds(lo, chunk)], idx_v)
            pltpu.async_copy(cb_sp.at[idx_v], rows_v, sem).wait()
            pltpu.sync_copy(rows_v, out_hbm.at[pl.ds(lo, chunk)])

    return k(codebook_flat, idxe)


@jax.jit
def kernel(z, codebook):
    zsq = jnp.sum(z ** 2, axis=1)
    csq = jnp.sum(codebook ** 2, axis=1)
    grid = _N_TOKENS // _BLK
    idx, idxe, part = pl.pallas_call(
        _vq_body,
        grid=(grid,),
        in_specs=[
            pl.BlockSpec((_BLK, _CODE_DIM), lambda i: (i, 0)),
            pl.BlockSpec((_NUM_CODES, _CODE_DIM), lambda i: (0, 0)),
            pl.BlockSpec((_BLK,), lambda i: (i,)),
            pl.BlockSpec((_NUM_CODES,), lambda i: (0,)),
        ],
        out_specs=[
            pl.BlockSpec((_BLK,), lambda i: (i,)),
            pl.BlockSpec((_BLK, _CODE_DIM), lambda i: (i, 0)),
            pl.BlockSpec((1, 1, 1), lambda i: (i, 0, 0)),
        ],
        out_shape=[
            jax.ShapeDtypeStruct((_N_TOKENS,), jnp.int32),
            jax.ShapeDtypeStruct((_N_TOKENS, _CODE_DIM), jnp.int32),
            jax.ShapeDtypeStruct((grid, 1, 1), jnp.float32),
        ],
        compiler_params=pltpu.CompilerParams(
            dimension_semantics=("parallel",)),
    )(z, codebook, zsq, csq)
    zq = _sc_gather(codebook.reshape(_NUM_CODES * _CODE_DIM),
                    idxe.reshape(_N_TOKENS * _CODE_DIM)
                    ).reshape(_N_TOKENS, _CODE_DIM)
    m = jnp.sum(part) / (_N_TOKENS * _CODE_DIM)
    loss = _BETA * m + m
    return (zq, idx, loss)
